# baseline (speedup 1.0000x reference)
import jax
import jax.numpy as jnp
from jax.experimental import pallas as pl

TILE_M = 256


def _router_tile(x_ref, w_ref, b_ref, o_ref):
    logits = jax.lax.dot_general(
        x_ref[...], w_ref[...],
        dimension_numbers=(((1,), (1,)), ((), ())),
        preferred_element_type=jnp.float32,
    ) + b_ref[...]
    m = jnp.max(logits, axis=-1, keepdims=True)
    e = jnp.exp(logits - m)
    o_ref[...] = e / jnp.sum(e, axis=-1, keepdims=True)


def kernel(x, W, b):
    B, T, D = x.shape
    E = W.shape[0]
    rows = B * T
    x2 = x.reshape(rows, D)
    grid = (rows // TILE_M,)
    out = pl.pallas_call(
        _router_tile,
        grid=grid,
        in_specs=[
            pl.BlockSpec((TILE_M, D), lambda i: (i, 0)),
            pl.BlockSpec((E, D), lambda i: (0, 0)),
            pl.BlockSpec((E,), lambda i: (0,)),
        ],
        out_specs=pl.BlockSpec((TILE_M, E), lambda i: (i, 0)),
        out_shape=jax.ShapeDtypeStruct((rows, E), jnp.float32),
    )(x2, W, b)
    return out.reshape(B, T, E)


# speedup vs baseline: 1.2284x; 1.2284x over previous
import jax
import jax.numpy as jnp
from jax.experimental import pallas as pl
from jax.experimental.pallas import tpu as pltpu

TILE_M = 1024


def _router_tile(x_ref, w_ref, b_ref, o_ref):
    logits = jax.lax.dot_general(
        x_ref[...], w_ref[...],
        dimension_numbers=(((1,), (1,)), ((), ())),
        preferred_element_type=jnp.float32,
    ) + b_ref[...]
    m = jnp.max(logits, axis=-1, keepdims=True)
    e = jnp.exp(logits - m)
    o_ref[...] = e / jnp.sum(e, axis=-1, keepdims=True)


def kernel(x, W, b):
    B, T, D = x.shape
    E = W.shape[0]
    rows = B * T
    x2 = x.reshape(rows, D)
    grid = (rows // TILE_M,)
    out = pl.pallas_call(
        _router_tile,
        grid=grid,
        in_specs=[
            pl.BlockSpec((TILE_M, D), lambda i: (i, 0)),
            pl.BlockSpec((E, D), lambda i: (0, 0)),
            pl.BlockSpec((E,), lambda i: (0,)),
        ],
        out_specs=pl.BlockSpec((TILE_M, E), lambda i: (i, 0)),
        out_shape=jax.ShapeDtypeStruct((rows, E), jnp.float32),
        compiler_params=pltpu.CompilerParams(
            dimension_semantics=("parallel",),
        ),
    )(x2, W, b)
    return out.reshape(B, T, E)


# speedup vs baseline: 1.2291x; 1.0006x over previous
import jax
import jax.numpy as jnp
from jax.experimental import pallas as pl
from jax.experimental.pallas import tpu as pltpu

TILE_M = 512


def _router_tile(x_ref, w_ref, b_ref, o_ref):
    logits = jax.lax.dot_general(
        x_ref[...], w_ref[...],
        dimension_numbers=(((1,), (1,)), ((), ())),
        preferred_element_type=jnp.float32,
    ) + b_ref[...]
    m = jnp.max(logits, axis=-1, keepdims=True)
    e = jnp.exp(logits - m)
    o_ref[...] = e / jnp.sum(e, axis=-1, keepdims=True)


def kernel(x, W, b):
    B, T, D = x.shape
    E = W.shape[0]
    rows = B * T
    x2 = x.reshape(rows, D)
    grid = (rows // TILE_M,)
    out = pl.pallas_call(
        _router_tile,
        grid=grid,
        in_specs=[
            pl.BlockSpec((TILE_M, D), lambda i: (i, 0)),
            pl.BlockSpec((E, D), lambda i: (0, 0)),
            pl.BlockSpec((E,), lambda i: (0,)),
        ],
        out_specs=pl.BlockSpec((TILE_M, E), lambda i: (i, 0)),
        out_shape=jax.ShapeDtypeStruct((rows, E), jnp.float32),
        compiler_params=pltpu.CompilerParams(
            dimension_semantics=("parallel",),
        ),
    )(x2, W, b)
    return out.reshape(B, T, E)
